# Initial kernel scaffold; baseline (speedup 1.0000x reference)
import functools
import jax
import jax.numpy as jnp
from jax import lax
from jax.experimental import pallas as pl
from jax.experimental.pallas import tpu as pltpu
from jax.experimental.pallas import tpu_sc as plsc

VOCAB = 1000

NC = 2
NS = 16
NW = NC * NS

B_TOT = 1024 * 50
PER_W = B_TOT // NW
CH = 64
N_CH = PER_W // CH

_mesh = plsc.VectorSubcoreMesh(
    core_axis_name="c", subcore_axis_name="s", num_cores=NC, num_subcores=NS
)


@functools.partial(
    pl.kernel,
    out_type=jax.ShapeDtypeStruct((B_TOT, VOCAB), jnp.float32),
    mesh=_mesh,
    scratch_types=[
        pltpu.VMEM((PER_W,), jnp.int32),
        pltpu.VMEM((CH, VOCAB), jnp.float32),
        pltpu.SemaphoreType.DMA,
    ],
)
def _sc_gather(table_hbm, idx_hbm, out_hbm, idx_v, buf, gsem):
    wid = lax.axis_index("s") * NC + lax.axis_index("c")
    base = wid * PER_W
    pltpu.sync_copy(idx_hbm.at[pl.ds(base, PER_W)], idx_v)

    @pl.loop(0, N_CH)
    def _chunk(c):
        off = c * CH
        pltpu.async_copy(
            table_hbm.at[idx_v.at[pl.ds(off, CH)]], buf, gsem
        ).wait()
        pltpu.sync_copy(buf, out_hbm.at[pl.ds(base + off, CH)])


def kernel(idx, table):
    B, T = idx.shape
    flat_idx = idx.reshape(-1).astype(jnp.int32)
    out = _sc_gather(table, flat_idx)
    return out.reshape(B, T, VOCAB)


# speedup vs baseline: 1.0080x; 1.0080x over previous
import functools
import jax
import jax.numpy as jnp
from jax import lax
from jax.experimental import pallas as pl
from jax.experimental.pallas import tpu as pltpu
from jax.experimental.pallas import tpu_sc as plsc

VOCAB = 1000
B, T = 1024, 50

NC = 2
NS = 16
NW = NC * NS
NB = B // NW

_mesh = plsc.VectorSubcoreMesh(
    core_axis_name="c", subcore_axis_name="s", num_cores=NC, num_subcores=NS
)


@functools.partial(
    pl.kernel,
    out_type=jax.ShapeDtypeStruct((B, T, VOCAB), jnp.float32),
    mesh=_mesh,
    scratch_types=[
        pltpu.VMEM((NB, T), jnp.int32),
        pltpu.VMEM((T, VOCAB), jnp.float32),
        pltpu.SemaphoreType.DMA,
    ],
    compiler_params=pltpu.CompilerParams(use_tc_tiling_on_sc=False),
)
def _sc_gather(table_hbm, idx_hbm, out_hbm, idx_v, buf, gsem):
    wid = lax.axis_index("s") * NC + lax.axis_index("c")
    b0 = wid * NB
    pltpu.sync_copy(idx_hbm.at[pl.ds(b0, NB)], idx_v)

    @pl.loop(0, NB)
    def _batch(b):
        pltpu.async_copy(table_hbm.at[idx_v.at[b]], buf, gsem).wait()
        pltpu.sync_copy(buf, out_hbm.at[b0 + b])


def kernel(idx, table):
    return _sc_gather(table, idx.astype(jnp.int32))
